# Initial kernel scaffold; baseline (speedup 1.0000x reference)
import jax
import jax.numpy as jnp
from jax.experimental import pallas as pl


def kernel(x, edge_index, Wl0, Wr0, b0, g0, be0, Wl1, Wr1, b1, g1, be1, Wl2, Wr2, b2, g2, be2):
    raise NotImplementedError("write your pallas kernel here")


# speedup vs baseline: 9.8248x; 9.8248x over previous
import functools

import jax
import jax.numpy as jnp
from jax import lax
from jax.experimental import pallas as pl
from jax.experimental.pallas import tpu as pltpu
from jax.experimental.pallas import tpu_sc as plsc

_N = 10000
_E = 320000
_D = 128
_EPS = 1e-5

_NC = 2
_NS = 16
_NW = _NC * _NS
_EPW = 10240
_EP = _NW * _EPW
_C = 128
_T = _EPW // _C
_CC = 128
_TC_ = _EPW // _CC
_RPT = 632
_R = _NS * _RPT
_BR = 400
_NB = _N // _BR


def _sc_mesh():
    return plsc.VectorSubcoreMesh(core_axis_name="c", subcore_axis_name="s",
                                  num_cores=_NC, num_subcores=_NS)


def _sc_agg_body(h_hbm, src_hbm, dst_hbm, z128,
                 part_out,
                 src_r, dst_r, buf0, buf1, sem0, sem1,
                 isem0, isem1, isem2, isem3, acc_sh):
    c = lax.axis_index("c")
    s = lax.axis_index("s")
    wid = s * _NC + c
    r0 = s * _RPT

    pltpu.sync_copy(z128.at[pl.ds(r0, _RPT)], acc_sh.at[pl.ds(r0, _RPT)])
    plsc.subcore_barrier()

    bufs = (buf0, buf1)
    sems = (sem0, sem1)
    isems = (isem0, isem1, isem2, isem3)

    def idx_copies(jj, slot, isem):
        return (pltpu.make_async_copy(src_hbm.at[wid, jj], src_r.at[slot], isem),
                pltpu.make_async_copy(dst_hbm.at[wid, jj], dst_r.at[slot], isem))

    for d in idx_copies(0, 0, isems[0]):
        d.start()
        d.wait()
    pltpu.make_async_copy(h_hbm.at[src_r.at[0]], buf0, sem0).start()
    for k in (1, 2, 3):
        for d in idx_copies(k, k, isems[k]):
            d.start()

    def step(i, _):
        j = i * 4
        for k in range(4):
            jj = j + k
            b = k % 2
            sn = (k + 1) % 4

            @pl.when(jj + 1 < _T)
            def _start_gather():
                for d in idx_copies(jj + 1, sn, isems[sn]):
                    d.wait()
                pltpu.make_async_copy(
                    h_hbm.at[src_r.at[sn]], bufs[1 - b], sems[1 - b]
                ).start()

            pltpu.make_async_copy(h_hbm.at[src_r.at[k]], bufs[b], sems[b]).wait()
            pltpu.sync_copy(bufs[b], acc_sh.at[dst_r.at[k]], add=True)

            @pl.when(jj + 4 < _T)
            def _prefetch_idx():
                for d in idx_copies(jj + 4, k, isems[k]):
                    d.start()
        return 0

    lax.fori_loop(0, _T // 4, step, 0)

    plsc.subcore_barrier()
    pltpu.sync_copy(acc_sh.at[pl.ds(r0, _RPT)], part_out.at[c, pl.ds(r0, _RPT)])


@functools.lru_cache(maxsize=None)
def _get_sc_agg():
    return pl.kernel(
        _sc_agg_body,
        out_type=(jax.ShapeDtypeStruct((_NC, _R, _D), jnp.float32),),
        mesh=_sc_mesh(),
        scratch_types=[
            pltpu.VMEM((4, _C), jnp.int32),
            pltpu.VMEM((4, _C), jnp.int32),
            pltpu.VMEM((_C, _D), jnp.float32),
            pltpu.VMEM((_C, _D), jnp.float32),
            pltpu.SemaphoreType.DMA,
            pltpu.SemaphoreType.DMA,
            pltpu.SemaphoreType.DMA,
            pltpu.SemaphoreType.DMA,
            pltpu.SemaphoreType.DMA,
            pltpu.SemaphoreType.DMA,
            pltpu.VMEM_SHARED((_R, _D), jnp.float32),
        ],
        name="sc_sage_agg",
    )


def _sc_cnt_body(dst_hbm, z128, o128,
                 cnt_out,
                 dst_r, ones_v, isem0, isem1, isem2, isem3, cnt_sh):
    c = lax.axis_index("c")
    s = lax.axis_index("s")
    wid = s * _NC + c
    r0 = s * _RPT

    pltpu.sync_copy(o128, ones_v)
    pltpu.sync_copy(z128.at[pl.ds(r0, _RPT)], cnt_sh.at[pl.ds(r0, _RPT)])
    plsc.subcore_barrier()

    isems = (isem0, isem1, isem2, isem3)
    d0 = pltpu.make_async_copy(dst_hbm.at[wid, 0], dst_r.at[0], isems[0])
    d0.start()
    d0.wait()
    for k in (1, 2, 3):
        pltpu.make_async_copy(dst_hbm.at[wid, k], dst_r.at[k], isems[k]).start()

    def step(i, _):
        j = i * 4
        for k in range(4):
            jj = j + k

            @pl.when(jj > 0)
            def _wait_idx():
                pltpu.make_async_copy(
                    dst_hbm.at[wid, jj], dst_r.at[k], isems[k]
                ).wait()

            pltpu.sync_copy(ones_v, cnt_sh.at[dst_r.at[k]], add=True)

            @pl.when(jj + 4 < _T)
            def _prefetch_idx():
                pltpu.make_async_copy(
                    dst_hbm.at[wid, jj + 4], dst_r.at[k], isems[k]
                ).start()
        return 0

    lax.fori_loop(0, _T // 4, step, 0)

    plsc.subcore_barrier()
    pltpu.sync_copy(cnt_sh.at[pl.ds(r0, _RPT)], cnt_out.at[c, pl.ds(r0, _RPT)])


@functools.lru_cache(maxsize=None)
def _get_sc_cnt():
    return pl.kernel(
        _sc_cnt_body,
        out_type=(jax.ShapeDtypeStruct((_NC, _R, _D), jnp.float32),),
        mesh=_sc_mesh(),
        scratch_types=[
            pltpu.VMEM((4, _C), jnp.int32),
            pltpu.VMEM((_C, _D), jnp.float32),
            pltpu.SemaphoreType.DMA,
            pltpu.SemaphoreType.DMA,
            pltpu.SemaphoreType.DMA,
            pltpu.SemaphoreType.DMA,
            pltpu.VMEM_SHARED((_R, _D), jnp.float32),
        ],
        name="sc_sage_cnt",
    )


def _tc_inv_body(cnt_ref, inv_ref):
    cnt = cnt_ref[0, :, :1] + cnt_ref[1, :, :1]
    inv = 1.0 / jnp.maximum(cnt, 1.0)
    inv_ref[:] = jnp.broadcast_to(inv, inv_ref.shape)


_tc_inv = pl.pallas_call(
    _tc_inv_body,
    grid=(_NB,),
    in_specs=[pl.BlockSpec((_NC, _BR, _D), lambda i: (0, i, 0))],
    out_specs=pl.BlockSpec((_BR, 8), lambda i: (i, 0)),
    out_shape=jax.ShapeDtypeStruct((_N, 8), jnp.float32),
)


def _tc_linear_body(part_ref, inv_ref, h_ref, wl_ref, wr_ref, b_ref,
                    z_ref, stats_ref, acc):
    i = pl.program_id(0)
    p = part_ref[0] + part_ref[1]
    mean = p * inv_ref[:, :1]
    z = (jnp.dot(mean, wl_ref[:], preferred_element_type=jnp.float32)
         + jnp.dot(h_ref[:], wr_ref[:], preferred_element_type=jnp.float32)
         + b_ref[:])
    z_ref[:] = z

    @pl.when(i == 0)
    def _init():
        acc[:] = jnp.zeros_like(acc)

    acc[0:1] = acc[0:1] + jnp.sum(z, axis=0, keepdims=True)
    acc[1:2] = acc[1:2] + jnp.sum(z * z, axis=0, keepdims=True)

    @pl.when(i == _NB - 1)
    def _flush():
        stats_ref[:] = acc[:]


_tc_linear = pl.pallas_call(
    _tc_linear_body,
    grid=(_NB,),
    in_specs=[
        pl.BlockSpec((_NC, _BR, _D), lambda i: (0, i, 0)),
        pl.BlockSpec((_BR, 8), lambda i: (i, 0)),
        pl.BlockSpec((_BR, _D), lambda i: (i, 0)),
        pl.BlockSpec((_D, _D), lambda i: (0, 0)),
        pl.BlockSpec((_D, _D), lambda i: (0, 0)),
        pl.BlockSpec((1, _D), lambda i: (0, 0)),
    ],
    out_specs=[
        pl.BlockSpec((_BR, _D), lambda i: (i, 0)),
        pl.BlockSpec((8, _D), lambda i: (0, 0)),
    ],
    out_shape=[
        jax.ShapeDtypeStruct((_N, _D), jnp.float32),
        jax.ShapeDtypeStruct((8, _D), jnp.float32),
    ],
    scratch_shapes=[pltpu.VMEM((8, _D), jnp.float32)],
)


def _tc_bn_relu_body(z_ref, stats_ref, g_ref, be_ref, out_ref):
    mu = stats_ref[0:1] * (1.0 / _N)
    ex2 = stats_ref[1:2] * (1.0 / _N)
    var = ex2 - mu * mu
    scale = g_ref[:] * lax.rsqrt(var + _EPS)
    out_ref[:] = jnp.maximum((z_ref[:] - mu) * scale + be_ref[:], 0.0)


_tc_bn_relu = pl.pallas_call(
    _tc_bn_relu_body,
    grid=(_NB,),
    in_specs=[
        pl.BlockSpec((_BR, _D), lambda i: (i, 0)),
        pl.BlockSpec((8, _D), lambda i: (0, 0)),
        pl.BlockSpec((1, _D), lambda i: (0, 0)),
        pl.BlockSpec((1, _D), lambda i: (0, 0)),
    ],
    out_specs=pl.BlockSpec((_BR, _D), lambda i: (i, 0)),
    out_shape=jax.ShapeDtypeStruct((_N, _D), jnp.float32),
)


def kernel(x, edge_index, Wl0, Wr0, b0, g0, be0, Wl1, Wr1, b1, g1, be1,
           Wl2, Wr2, b2, g2, be2):
    src = edge_index[0].astype(jnp.int32)
    dst = edge_index[1].astype(jnp.int32)

    npad = _EP - _E
    pad_ids = jnp.arange(npad, dtype=jnp.int32)
    src_flat = jnp.concatenate([src, (pad_ids * 37) % _N])
    dst_flat = jnp.concatenate([dst, _N + pad_ids % (_R - _N)])
    src_p = src_flat.reshape(_NW, _T, _C)
    dst_p = dst_flat.reshape(_NW, _T, _C)

    z128 = jnp.zeros((_R, _D), jnp.float32)
    o128 = jnp.ones((_C, _D), jnp.float32)

    (cnt,) = _get_sc_cnt()(dst_p, z128, o128)
    inv = _tc_inv(cnt)

    params = [(Wl0, Wr0, b0, g0, be0), (Wl1, Wr1, b1, g1, be1),
              (Wl2, Wr2, b2, g2, be2)]
    h = x
    for (Wl, Wr, b, g, be) in params:
        (part,) = _get_sc_agg()(h, src_p, dst_p, z128)
        z, stats = _tc_linear(part, inv, h, Wl, Wr, b.reshape(1, _D))
        h = _tc_bn_relu(z, stats, g.reshape(1, _D), be.reshape(1, _D))
    return h
